# baseline (speedup 1.0000x reference)
import functools

import jax
import jax.numpy as jnp
from jax import lax
from jax.experimental import pallas as pl
from jax.experimental.pallas import tpu as pltpu
from jax.experimental.pallas import tpu_sc as plsc

T = 4096
H = 1024
F = 4096
E = 8
KTOP = 2
G = E + 1
BT = 256
P = T * KTOP + T + G * BT
NB = P // BT
FC = 512
NFC = F // FC

NC, NS = 2, 16
NW = NC * NS
BPW = P // NW
GC = 24
NGC = BPW // GC
RPW = T // NW
CC = 16
NCC = RPW // CC

@functools.lru_cache(maxsize=None)
def _sc_mesh():
    return plsc.VectorSubcoreMesh(core_axis_name="c", subcore_axis_name="s")


def _router_body(x_ref, wr_ref, e_ref, w_ref):
    logits = lax.dot_general(x_ref[...], wr_ref[...], (((1,), (1,)), ((), ())),
                             preferred_element_type=jnp.float32)
    iota = lax.broadcasted_iota(jnp.int32, logits.shape, 1)
    m0 = jnp.max(logits, axis=1, keepdims=True)
    i0 = jnp.min(jnp.where(logits == m0, iota, E), axis=1, keepdims=True)
    masked = jnp.where(iota == i0, -jnp.inf, logits)
    m1 = jnp.max(masked, axis=1, keepdims=True)
    i1 = jnp.min(jnp.where(masked == m1, iota, E), axis=1, keepdims=True)
    w0 = 1.0 / (1.0 + jnp.exp(m1 - m0))
    e_ref[...] = jnp.concatenate([i0, i1], axis=1)
    w_ref[...] = jnp.concatenate([w0, 1.0 - w0], axis=1)


def _router(x, Wr):
    return pl.pallas_call(
        _router_body,
        out_shape=(jax.ShapeDtypeStruct((T, KTOP), jnp.int32),
                   jax.ShapeDtypeStruct((T, KTOP), jnp.float32)),
    )(x, Wr)


def _dispatch(e2, w2):
    ef = jnp.concatenate([e2.reshape(-1), jnp.full((T,), E, jnp.int32)])
    wf = jnp.concatenate([w2.reshape(-1), jnp.ones((T,), jnp.float32)])
    tok = jnp.concatenate([jnp.arange(T * KTOP, dtype=jnp.int32) // KTOP,
                           jnp.arange(T, dtype=jnp.int32)])
    onehot = (ef[:, None] == jnp.arange(G, dtype=jnp.int32)[None, :]).astype(jnp.int32)
    ranks = jnp.cumsum(onehot, axis=0) - 1
    counts = jnp.sum(onehot, axis=0)
    padded = ((counts + BT - 1) // BT) * BT
    offs = jnp.concatenate([jnp.zeros((1,), jnp.int32),
                            jnp.cumsum(padded)])
    dest = offs[ef] + jnp.sum(onehot * ranks, axis=1)
    gtok = jnp.zeros((P,), jnp.int32).at[dest].set(tok)
    gw = jnp.zeros((P,), jnp.float32).at[dest].set(wf)
    nvalid = offs[G] // BT
    bstart = jnp.arange(NB, dtype=jnp.int32) * BT
    be = jnp.searchsorted(offs[:G], bstart, side="right").astype(jnp.int32) - 1
    be = jnp.minimum(be, G - 1)
    dp = dest[: T * KTOP].reshape(T, KTOP)
    d0, d1 = dp[:, 0], dp[:, 1]
    ds = dest[T * KTOP:]
    return gtok, gw, be, nvalid, d0, d1, ds


def _gather_body(x_hbm, gtok_hbm, xs_hbm, idx_v, b0_v, b1_v,
                 gs0, gs1, ss0, ss1):
    wid = lax.axis_index("s") * NC + lax.axis_index("c")
    base = wid * BPW
    pltpu.sync_copy(gtok_hbm.at[pl.ds(base, BPW)], idx_v)
    bufs = (b0_v, b1_v)
    gsems = (gs0, gs1)
    ssems = (ss0, ss1)

    def g_copy(k):
        b = k % 2
        return pltpu.make_async_copy(
            x_hbm.at[idx_v.at[pl.ds(k * GC, GC)]], bufs[b], gsems[b])

    def s_copy(k):
        b = k % 2
        return pltpu.make_async_copy(
            bufs[b], xs_hbm.at[pl.ds(base + k * GC, GC)], ssems[b])

    g_copy(0).start()
    for k in range(NGC):
        g_copy(k).wait()
        s_copy(k).start()
        if k + 1 < NGC:
            if k >= 1:
                s_copy(k - 1).wait()
            g_copy(k + 1).start()
    s_copy(NGC - 2).wait()
    s_copy(NGC - 1).wait()


@functools.lru_cache(maxsize=None)
def _gather():
    return pl.kernel(
        _gather_body,
        out_type=jax.ShapeDtypeStruct((P, H), jnp.float32),
        mesh=_sc_mesh(),
        scratch_types=[pltpu.VMEM((BPW,), jnp.int32),
                       pltpu.VMEM((GC, H), jnp.float32),
                       pltpu.VMEM((GC, H), jnp.float32),
                       pltpu.SemaphoreType.DMA,
                       pltpu.SemaphoreType.DMA,
                       pltpu.SemaphoreType.DMA,
                       pltpu.SemaphoreType.DMA],
    )


def _mlp_body(be_ref, nv_ref, xs_ref, wg_ref, wu_ref, wd_ref, gw_ref, y_ref):
    i = pl.program_id(0)

    @pl.when(i < nv_ref[0])
    def _():
        xb = xs_ref[...].astype(jnp.bfloat16)
        acc = jnp.zeros((BT, H), jnp.float32)
        for c in range(NFC):
            wg_c = wg_ref[0, c * FC:(c + 1) * FC, :]
            wu_c = wu_ref[0, c * FC:(c + 1) * FC, :]
            wd_c = wd_ref[0, :, c * FC:(c + 1) * FC]
            g = lax.dot_general(xb, wg_c, (((1,), (1,)), ((), ())),
                                preferred_element_type=jnp.float32)
            u = lax.dot_general(xb, wu_c, (((1,), (1,)), ((), ())),
                                preferred_element_type=jnp.float32)
            hc = (g / (1.0 + jnp.exp(-g)) * u).astype(jnp.bfloat16)
            acc = acc + lax.dot_general(hc, wd_c, (((1,), (1,)), ((), ())),
                                        preferred_element_type=jnp.float32)
        y_ref[...] = acc * gw_ref[0]


def _grouped_mlp(be, nvalid, xs, WgA, WuA, WdA, gw3):
    grid_spec = pltpu.PrefetchScalarGridSpec(
        num_scalar_prefetch=2,
        grid=(NB,),
        in_specs=[
            pl.BlockSpec((BT, H), lambda i, be, nv: (i, 0)),
            pl.BlockSpec((1, F, H), lambda i, be, nv: (be[i], 0, 0)),
            pl.BlockSpec((1, F, H), lambda i, be, nv: (be[i], 0, 0)),
            pl.BlockSpec((1, H, F), lambda i, be, nv: (be[i], 0, 0)),
            pl.BlockSpec((1, BT, 1), lambda i, be, nv: (i, 0, 0)),
        ],
        out_specs=pl.BlockSpec((BT, H), lambda i, be, nv: (i, 0)),
    )
    return pl.pallas_call(
        _mlp_body,
        grid_spec=grid_spec,
        out_shape=jax.ShapeDtypeStruct((P, H), jnp.float32),
        compiler_params=pltpu.CompilerParams(
            dimension_semantics=("arbitrary",)),
    )(be, nvalid, xs, WgA, WuA, WdA, gw3)


def _combine_body(y_hbm, d0_hbm, d1_hbm, ds_hbm, out_hbm,
                  i0_v, i1_v, is_v, a_v, b_v, c_v, o_v):
    wid = lax.axis_index("s") * NC + lax.axis_index("c")
    base = wid * RPW
    pltpu.sync_copy(d0_hbm.at[pl.ds(base, RPW)], i0_v)
    pltpu.sync_copy(d1_hbm.at[pl.ds(base, RPW)], i1_v)
    pltpu.sync_copy(ds_hbm.at[pl.ds(base, RPW)], is_v)
    for k in range(NCC):
        pltpu.sync_copy(y_hbm.at[i0_v.at[pl.ds(k * CC, CC)]], a_v)
        pltpu.sync_copy(y_hbm.at[i1_v.at[pl.ds(k * CC, CC)]], b_v)
        pltpu.sync_copy(y_hbm.at[is_v.at[pl.ds(k * CC, CC)]], c_v)

        def body(r, _):
            def inner(c, _):
                sl = pl.ds(c * 16, 16)
                o_v[r, sl] = a_v[r, sl] + b_v[r, sl] + c_v[r, sl]
                return 0
            return lax.fori_loop(0, H // 16, inner, 0)
        lax.fori_loop(0, CC, body, 0)
        pltpu.sync_copy(o_v, out_hbm.at[pl.ds(base + k * CC, CC)])


@functools.lru_cache(maxsize=None)
def _combine():
    return pl.kernel(
        _combine_body,
        out_type=jax.ShapeDtypeStruct((T, H), jnp.float32),
        mesh=_sc_mesh(),
        scratch_types=[pltpu.VMEM((RPW,), jnp.int32),
                       pltpu.VMEM((RPW,), jnp.int32),
                       pltpu.VMEM((RPW,), jnp.int32),
                       pltpu.VMEM((CC, H), jnp.float32),
                       pltpu.VMEM((CC, H), jnp.float32),
                       pltpu.VMEM((CC, H), jnp.float32),
                       pltpu.VMEM((CC, H), jnp.float32)],
    )


def kernel(x, Wr, Wg, Wu, Wd, Wgs, Wus, Wds):
    x2 = x.reshape(T, H)
    WgA = jnp.concatenate([Wg, Wgs[None]], axis=0).astype(jnp.bfloat16)
    WuA = jnp.concatenate([Wu, Wus[None]], axis=0).astype(jnp.bfloat16)
    WdA = jnp.concatenate([Wd, Wds[None]], axis=0).astype(jnp.bfloat16)

    e2, w2 = _router(x2, Wr)
    gtok, gw, be, nvalid, d0, d1, ds = _dispatch(e2, w2)

    xs = _gather()(x2, gtok)
    gw3 = gw.reshape(NB, BT, 1)
    y = _grouped_mlp(be, nvalid.reshape(1), xs, WgA, WuA, WdA, gw3)
    out = _combine()(y, d0, d1, ds)
    return out.reshape(x.shape)


# speedup vs baseline: 1.0925x; 1.0925x over previous
import functools

import jax
import jax.numpy as jnp
from jax import lax
from jax.experimental import pallas as pl
from jax.experimental.pallas import tpu as pltpu
from jax.experimental.pallas import tpu_sc as plsc

T = 4096
H = 1024
F = 4096
E = 8
KTOP = 2
G = E + 1
BT = 256
P = T * KTOP + T + G * BT
NB = P // BT
FC = 512
NFC = F // FC

NC, NS = 2, 16
NW = NC * NS
BPW = P // NW
GC = 24
NGC = BPW // GC
RPW = T // NW
CC = 16
NCC = RPW // CC

@functools.lru_cache(maxsize=None)
def _sc_mesh():
    return plsc.VectorSubcoreMesh(core_axis_name="c", subcore_axis_name="s")


CPW = (T * KTOP) // NW
SPW = T // NW


def _router_body(x_ref, wr_ref, e_ref, w_ref, base_ref, o8_ref, be_ref, nv_ref):
    logits = lax.dot_general(x_ref[...], wr_ref[...], (((1,), (1,)), ((), ())),
                             preferred_element_type=jnp.float32)
    iota = lax.broadcasted_iota(jnp.int32, logits.shape, 1)
    m0 = jnp.max(logits, axis=1, keepdims=True)
    i0 = jnp.min(jnp.where(logits == m0, iota, E), axis=1, keepdims=True)
    masked = jnp.where(iota == i0, -jnp.inf, logits)
    m1 = jnp.max(masked, axis=1, keepdims=True)
    i1 = jnp.min(jnp.where(masked == m1, iota, E), axis=1, keepdims=True)
    w0 = 1.0 / (1.0 + jnp.exp(m1 - m0))
    e_ref[...] = jnp.concatenate([i0, i1], axis=1)
    w_ref[...] = jnp.concatenate([w0, 1.0 - w0], axis=1)

    NCH = NW * 16
    ioE = lax.broadcasted_iota(jnp.int32, (T, E), 1)
    oh = ((i0 == ioE).astype(jnp.float32) + (i1 == ioE).astype(jnp.float32))
    tw = lax.broadcasted_iota(jnp.int32, (NCH, T), 0)
    tt = lax.broadcasted_iota(jnp.int32, (NCH, T), 1) // (T // NCH)
    sel = (tw == tt).astype(jnp.float32)
    ch = lax.dot_general(sel, oh, (((1,), (0,)), ((), ())),
                         preferred_element_type=jnp.float32)
    lw = lax.broadcasted_iota(jnp.int32, (NCH, NCH), 0)
    lc = lax.broadcasted_iota(jnp.int32, (NCH, NCH), 1)
    ltri = (lc < lw).astype(jnp.float32)
    pre = lax.dot_general(ltri, ch, (((1,), (0,)), ((), ())),
                          preferred_element_type=jnp.float32)
    counts = jnp.sum(ch, axis=0, keepdims=True)
    padded = jnp.ceil(counts / BT) * BT
    le = lax.broadcasted_iota(jnp.int32, (E, E), 0)
    lf = lax.broadcasted_iota(jnp.int32, (E, E), 1)
    mup = (le < lf).astype(jnp.float32)
    offsE = lax.dot_general(padded, mup, (((1,), (0,)), ((), ())),
                            preferred_element_type=jnp.float32)
    offs8 = jnp.sum(padded, axis=1, keepdims=True)
    base_ref[...] = (offsE + pre).astype(jnp.int32)
    o8_ref[...] = offs8.astype(jnp.int32)

    offs9 = jnp.concatenate([offsE, offs8], axis=1)
    bstart = (lax.broadcasted_iota(jnp.int32, (1, 64), 1) * BT).astype(jnp.float32)
    ge = jnp.zeros((1, 64), jnp.float32)
    for g in range(G):
        ge = ge + (bstart >= offs9[0, g]).astype(jnp.float32)
    be_ref[...] = (ge - 1.0).astype(jnp.int32)
    nv_ref[...] = ((offs8 + T) / BT).astype(jnp.int32)


def _router(x, Wr):
    return pl.pallas_call(
        _router_body,
        out_shape=(jax.ShapeDtypeStruct((T, KTOP), jnp.int32),
                   jax.ShapeDtypeStruct((T, KTOP), jnp.float32),
                   jax.ShapeDtypeStruct((NW * 16, E), jnp.int32),
                   jax.ShapeDtypeStruct((1, 1), jnp.int32),
                   jax.ShapeDtypeStruct((1, 64), jnp.int32),
                   jax.ShapeDtypeStruct((1, 1), jnp.int32)),
    )(x, Wr)


def _dispatch_body(efT_hbm, wfT_hbm, basex_hbm, b8x_hbm,
                   d0_hbm, d1_hbm, ds_hbm, tk_hbm, gtok_hbm, gw_hbm,
                   bx_v, ev_v, wv_v, b8s_v, db_v, tokb_v, gwb_v,
                   d0b_v, d1b_v, dsb_v, tkb_v, onesb_v):
    wid = lax.axis_index("s") * NC + lax.axis_index("c")
    pltpu.sync_copy(basex_hbm.at[pl.ds(wid, 1)], bx_v)
    pltpu.sync_copy(efT_hbm.at[pl.ds(wid, 1)], ev_v)
    pltpu.sync_copy(wfT_hbm.at[pl.ds(wid, 1)], wv_v)
    pltpu.sync_copy(b8x_hbm.at[pl.ds(wid, 1)], b8s_v)
    iota16 = lax.broadcasted_iota(jnp.int32, (16,), 0)
    one = jnp.ones((16,), jnp.int32)
    ctr = [bx_v[0, e, :] for e in range(E)]
    dsts = []
    for j in range(16):
        ev = ev_v[0, j, :]
        dst = jnp.zeros((16,), jnp.int32)
        for e in range(E):
            m = ev == e
            dst = jnp.where(m, ctr[e], dst)
            ctr[e] = ctr[e] + jnp.where(m, one, 0)
        dsts.append(dst)
        db_v[j // 8, pl.ds((j % 8) * 16, 16)] = dst
        tokb_v[j // 8, pl.ds((j % 8) * 16, 16)] = wid * SPW + 8 * iota16 + (j // 2)
        gwb_v[j // 8, pl.ds((j % 8) * 16, 16)] = wv_v[0, j, :]
    b8v = b8s_v[0, :]
    for m in range(8):
        d0b_v[0, pl.ds(m * 16, 16)] = dsts[2 * m]
        d1b_v[0, pl.ds(m * 16, 16)] = dsts[2 * m + 1]
        dsb_v[0, pl.ds(m * 16, 16)] = b8v + m * 16 + iota16
        tkb_v[0, pl.ds(m * 16, 16)] = wid * SPW + 8 * iota16 + m
        onesb_v[0, pl.ds(m * 16, 16)] = jnp.ones((16,), jnp.float32)
    pltpu.sync_copy(d0b_v, d0_hbm.at[pl.ds(wid, 1)])
    pltpu.sync_copy(d1b_v, d1_hbm.at[pl.ds(wid, 1)])
    pltpu.sync_copy(dsb_v, ds_hbm.at[pl.ds(wid, 1)])
    pltpu.sync_copy(tkb_v, tk_hbm.at[pl.ds(wid, 1)])
    for j in range(KTOP):
        pltpu.sync_copy(tokb_v.at[j], gtok_hbm.at[db_v.at[j]])
        pltpu.sync_copy(gwb_v.at[j], gw_hbm.at[db_v.at[j]])
    pltpu.sync_copy(tkb_v.at[0], gtok_hbm.at[dsb_v.at[0]])
    pltpu.sync_copy(onesb_v.at[0], gw_hbm.at[dsb_v.at[0]])


@functools.lru_cache(maxsize=None)
def _dispatch_sc():
    return pl.kernel(
        _dispatch_body,
        out_type=(jax.ShapeDtypeStruct((NW, SPW), jnp.int32),
                  jax.ShapeDtypeStruct((NW, SPW), jnp.int32),
                  jax.ShapeDtypeStruct((NW, SPW), jnp.int32),
                  jax.ShapeDtypeStruct((NW, SPW), jnp.int32),
                  jax.ShapeDtypeStruct((P,), jnp.int32),
                  jax.ShapeDtypeStruct((P,), jnp.float32)),
        mesh=_sc_mesh(),
        scratch_types=[pltpu.VMEM((1, E, 16), jnp.int32),
                       pltpu.VMEM((1, 16, 16), jnp.int32),
                       pltpu.VMEM((1, 16, 16), jnp.float32),
                       pltpu.VMEM((1, 16), jnp.int32),
                       pltpu.VMEM((KTOP, 128), jnp.int32),
                       pltpu.VMEM((KTOP, 128), jnp.int32),
                       pltpu.VMEM((KTOP, 128), jnp.float32),
                       pltpu.VMEM((1, SPW), jnp.int32),
                       pltpu.VMEM((1, SPW), jnp.int32),
                       pltpu.VMEM((1, SPW), jnp.int32),
                       pltpu.VMEM((1, SPW), jnp.int32),
                       pltpu.VMEM((1, SPW), jnp.float32)],
    )


def _gather_body(x_hbm, gtok_hbm, xs_hbm, idx_v, b0_v, b1_v,
                 gs0, gs1, ss0, ss1):
    wid = lax.axis_index("s") * NC + lax.axis_index("c")
    base = wid * BPW
    pltpu.sync_copy(gtok_hbm.at[pl.ds(base, BPW)], idx_v)
    for j in range(BPW // 16 + 1):
        sl = pl.ds(min(j * 16, BPW - 16), 16)
        idx_v[sl] = jnp.clip(idx_v[sl], 0, T - 1)
    bufs = (b0_v, b1_v)
    gsems = (gs0, gs1)
    ssems = (ss0, ss1)

    def g_copy(k):
        b = k % 2
        return pltpu.make_async_copy(
            x_hbm.at[idx_v.at[pl.ds(k * GC, GC)]], bufs[b], gsems[b])

    def s_copy(k):
        b = k % 2
        return pltpu.make_async_copy(
            bufs[b], xs_hbm.at[pl.ds(base + k * GC, GC)], ssems[b])

    g_copy(0).start()
    for k in range(NGC):
        g_copy(k).wait()
        s_copy(k).start()
        if k + 1 < NGC:
            if k >= 1:
                s_copy(k - 1).wait()
            g_copy(k + 1).start()
    s_copy(NGC - 2).wait()
    s_copy(NGC - 1).wait()


@functools.lru_cache(maxsize=None)
def _gather():
    return pl.kernel(
        _gather_body,
        out_type=jax.ShapeDtypeStruct((P, H), jnp.float32),
        mesh=_sc_mesh(),
        scratch_types=[pltpu.VMEM((BPW,), jnp.int32),
                       pltpu.VMEM((GC, H), jnp.float32),
                       pltpu.VMEM((GC, H), jnp.float32),
                       pltpu.SemaphoreType.DMA,
                       pltpu.SemaphoreType.DMA,
                       pltpu.SemaphoreType.DMA,
                       pltpu.SemaphoreType.DMA],
    )


def _mlp_body(be_ref, nv_ref, xs_ref, wg_ref, wu_ref, wd_ref, gw_ref, y_ref):
    i = pl.program_id(0)

    @pl.when(i < nv_ref[0])
    def _():
        xb = xs_ref[...].astype(jnp.bfloat16)
        acc = jnp.zeros((BT, H), jnp.float32)
        for c in range(NFC):
            wg_c = wg_ref[0, c * FC:(c + 1) * FC, :]
            wu_c = wu_ref[0, c * FC:(c + 1) * FC, :]
            wd_c = wd_ref[0, :, c * FC:(c + 1) * FC]
            g = lax.dot_general(xb, wg_c, (((1,), (1,)), ((), ())),
                                preferred_element_type=jnp.float32)
            u = lax.dot_general(xb, wu_c, (((1,), (1,)), ((), ())),
                                preferred_element_type=jnp.float32)
            hc = (g / (1.0 + jnp.exp(-g)) * u).astype(jnp.bfloat16)
            acc = acc + lax.dot_general(hc, wd_c, (((1,), (1,)), ((), ())),
                                        preferred_element_type=jnp.float32)
        y_ref[...] = acc * gw_ref[0]


def _grouped_mlp(be, nvalid, xs, WgA, WuA, WdA, gw3):
    grid_spec = pltpu.PrefetchScalarGridSpec(
        num_scalar_prefetch=2,
        grid=(NB,),
        in_specs=[
            pl.BlockSpec((BT, H), lambda i, be, nv: (i, 0)),
            pl.BlockSpec((1, F, H), lambda i, be, nv: (be[i], 0, 0)),
            pl.BlockSpec((1, F, H), lambda i, be, nv: (be[i], 0, 0)),
            pl.BlockSpec((1, H, F), lambda i, be, nv: (be[i], 0, 0)),
            pl.BlockSpec((1, BT, 1), lambda i, be, nv: (i, 0, 0)),
        ],
        out_specs=pl.BlockSpec((BT, H), lambda i, be, nv: (i, 0)),
    )
    return pl.pallas_call(
        _mlp_body,
        grid_spec=grid_spec,
        out_shape=jax.ShapeDtypeStruct((P, H), jnp.float32),
        compiler_params=pltpu.CompilerParams(
            dimension_semantics=("arbitrary",)),
    )(be, nvalid, xs, WgA, WuA, WdA, gw3)


def _combine_body(y_hbm, d0_hbm, d1_hbm, ds_hbm, tk_hbm, out_hbm,
                  i0_v, i1_v, is_v, tk_v, a_v, b_v, c_v, o_v):
    wid = lax.axis_index("s") * NC + lax.axis_index("c")
    base = wid * RPW
    pltpu.sync_copy(d0_hbm.at[pl.ds(base, RPW)], i0_v)
    pltpu.sync_copy(d1_hbm.at[pl.ds(base, RPW)], i1_v)
    pltpu.sync_copy(ds_hbm.at[pl.ds(base, RPW)], is_v)
    pltpu.sync_copy(tk_hbm.at[pl.ds(wid * NCC, NCC)], tk_v)
    for k in range(NCC):
        pltpu.sync_copy(y_hbm.at[i0_v.at[pl.ds(k * CC, CC)]], a_v)
        pltpu.sync_copy(y_hbm.at[i1_v.at[pl.ds(k * CC, CC)]], b_v)
        pltpu.sync_copy(y_hbm.at[is_v.at[pl.ds(k * CC, CC)]], c_v)

        def body(r, _):
            def inner(c, _):
                sl = pl.ds(c * 16, 16)
                o_v[r, sl] = a_v[r, sl] + b_v[r, sl] + c_v[r, sl]
                return 0
            return lax.fori_loop(0, H // 16, inner, 0)
        lax.fori_loop(0, CC, body, 0)
        pltpu.sync_copy(o_v, out_hbm.at[tk_v.at[k]])


@functools.lru_cache(maxsize=None)
def _combine():
    return pl.kernel(
        _combine_body,
        out_type=jax.ShapeDtypeStruct((T, H), jnp.float32),
        mesh=_sc_mesh(),
        scratch_types=[pltpu.VMEM((RPW,), jnp.int32),
                       pltpu.VMEM((RPW,), jnp.int32),
                       pltpu.VMEM((RPW,), jnp.int32),
                       pltpu.VMEM((NCC, CC), jnp.int32),
                       pltpu.VMEM((CC, H), jnp.float32),
                       pltpu.VMEM((CC, H), jnp.float32),
                       pltpu.VMEM((CC, H), jnp.float32),
                       pltpu.VMEM((CC, H), jnp.float32)],
    )


def kernel(x, Wr, Wg, Wu, Wd, Wgs, Wus, Wds):
    x2 = x.reshape(T, H)
    WgA = jnp.concatenate([Wg, Wgs[None]], axis=0).astype(jnp.bfloat16)
    WuA = jnp.concatenate([Wu, Wus[None]], axis=0).astype(jnp.bfloat16)
    WdA = jnp.concatenate([Wd, Wds[None]], axis=0).astype(jnp.bfloat16)

    e2, w2, base512, o8, beP, nv = _router(x2, Wr)

    efT = e2.reshape(NW, 16, 8, KTOP).transpose(0, 2, 3, 1).reshape(NW, 16, 16)
    wfT = w2.reshape(NW, 16, 8, KTOP).transpose(0, 2, 3, 1).reshape(NW, 16, 16)
    basex2 = base512.reshape(NW, 16, E).transpose(0, 2, 1)
    b8 = o8.reshape(1) + SPW * jnp.arange(NW, dtype=jnp.int32)
    b8x = jnp.broadcast_to(b8[:, None], (NW, 16))

    d0, d1, ds, tok, gtok, gw = _dispatch_sc()(efT, wfT, basex2, b8x)
    d0, d1, ds, tok = (a.reshape(T) for a in (d0, d1, ds, tok))

    xs = _gather()(x2, gtok)
    gw3 = gw.reshape(NB, BT, 1)
    y = _grouped_mlp(beP.reshape(64), nv.reshape(1), xs, WgA, WuA, WdA, gw3)
    out = _combine()(y, d0, d1, ds, tok.reshape(T // CC, CC))
    return out.reshape(x.shape)


# speedup vs baseline: 1.3355x; 1.2225x over previous
import functools

import jax
import jax.numpy as jnp
from jax import lax
from jax.experimental import pallas as pl
from jax.experimental.pallas import tpu as pltpu
from jax.experimental.pallas import tpu_sc as plsc

T = 4096
H = 1024
F = 4096
E = 8
KTOP = 2
G = E + 1
BT = 256
P = T * KTOP + E * BT
NB = P // BT
BTS = 256
FC = 512
NFC = F // FC

NC, NS = 2, 16
NW = NC * NS
BPW = P // NW
GC = 32
NGC = BPW // GC
RPW = T // NW
CC = 16
NCC = RPW // CC

@functools.lru_cache(maxsize=None)
def _sc_mesh():
    return plsc.VectorSubcoreMesh(core_axis_name="c", subcore_axis_name="s")


CPW = (T * KTOP) // NW
SPW = T // NW


def _router_body(x_ref, wr_ref, e_ref, w_ref, base_ref, o8_ref, be_ref, nv_ref):
    logits = lax.dot_general(x_ref[...], wr_ref[...], (((1,), (1,)), ((), ())),
                             preferred_element_type=jnp.float32)
    iota = lax.broadcasted_iota(jnp.int32, logits.shape, 1)
    m0 = jnp.max(logits, axis=1, keepdims=True)
    i0 = jnp.min(jnp.where(logits == m0, iota, E), axis=1, keepdims=True)
    masked = jnp.where(iota == i0, -jnp.inf, logits)
    m1 = jnp.max(masked, axis=1, keepdims=True)
    i1 = jnp.min(jnp.where(masked == m1, iota, E), axis=1, keepdims=True)
    w0 = 1.0 / (1.0 + jnp.exp(m1 - m0))
    e_ref[...] = jnp.concatenate([i0, i1], axis=1)
    w_ref[...] = jnp.concatenate([w0, 1.0 - w0], axis=1)

    NCH = NW * 16
    ioE = lax.broadcasted_iota(jnp.int32, (T, E), 1)
    oh = ((i0 == ioE).astype(jnp.float32) + (i1 == ioE).astype(jnp.float32))
    tw = lax.broadcasted_iota(jnp.int32, (NCH, T), 0)
    tt = lax.broadcasted_iota(jnp.int32, (NCH, T), 1) // (T // NCH)
    sel = (tw == tt).astype(jnp.float32)
    ch = lax.dot_general(sel, oh, (((1,), (0,)), ((), ())),
                         preferred_element_type=jnp.float32)
    lw = lax.broadcasted_iota(jnp.int32, (NCH, NCH), 0)
    lc = lax.broadcasted_iota(jnp.int32, (NCH, NCH), 1)
    ltri = (lc < lw).astype(jnp.float32)
    pre = lax.dot_general(ltri, ch, (((1,), (0,)), ((), ())),
                          preferred_element_type=jnp.float32)
    counts = jnp.sum(ch, axis=0, keepdims=True)
    padded = jnp.ceil(counts / BT) * BT
    le = lax.broadcasted_iota(jnp.int32, (E, E), 0)
    lf = lax.broadcasted_iota(jnp.int32, (E, E), 1)
    mup = (le < lf).astype(jnp.float32)
    offsE = lax.dot_general(padded, mup, (((1,), (0,)), ((), ())),
                            preferred_element_type=jnp.float32)
    offs8 = jnp.sum(padded, axis=1, keepdims=True)
    base_ref[...] = (offsE + pre).astype(jnp.int32)
    o8_ref[...] = offs8.astype(jnp.int32)

    bstart = (lax.broadcasted_iota(jnp.int32, (1, 64), 1) * BT).astype(jnp.float32)
    ge = jnp.zeros((1, 64), jnp.float32)
    for g in range(1, E):
        ge = ge + (bstart >= offsE[0, g]).astype(jnp.float32)
    be_ref[...] = ge.astype(jnp.int32)
    nv_ref[...] = (offs8 / BT).astype(jnp.int32)


def _router(x, Wr):
    return pl.pallas_call(
        _router_body,
        out_shape=(jax.ShapeDtypeStruct((T, KTOP), jnp.int32),
                   jax.ShapeDtypeStruct((T, KTOP), jnp.float32),
                   jax.ShapeDtypeStruct((NW * 16, E), jnp.int32),
                   jax.ShapeDtypeStruct((1, 1), jnp.int32),
                   jax.ShapeDtypeStruct((1, 64), jnp.int32),
                   jax.ShapeDtypeStruct((1, 1), jnp.int32)),
    )(x, Wr)


def _dispatch_body(efT_hbm, wfT_hbm, basex_hbm,
                   d0_hbm, d1_hbm, tk_hbm, gtok_hbm, gw_hbm,
                   bx_v, ev_v, wv_v, db_v, tokb_v, gwb_v,
                   d0b_v, d1b_v, tkb_v):
    wid = lax.axis_index("s") * NC + lax.axis_index("c")
    pltpu.sync_copy(basex_hbm.at[pl.ds(wid, 1)], bx_v)
    pltpu.sync_copy(efT_hbm.at[pl.ds(wid, 1)], ev_v)
    pltpu.sync_copy(wfT_hbm.at[pl.ds(wid, 1)], wv_v)
    iota16 = lax.broadcasted_iota(jnp.int32, (16,), 0)
    one = jnp.ones((16,), jnp.int32)
    ctr = [bx_v[0, e, :] for e in range(E)]
    dsts = []
    for j in range(16):
        ev = ev_v[0, j, :]
        dst = jnp.zeros((16,), jnp.int32)
        for e in range(E):
            m = ev == e
            dst = jnp.where(m, ctr[e], dst)
            ctr[e] = ctr[e] + jnp.where(m, one, 0)
        dsts.append(dst)
        db_v[j // 8, pl.ds((j % 8) * 16, 16)] = dst
        tokb_v[j // 8, pl.ds((j % 8) * 16, 16)] = wid * SPW + 8 * iota16 + (j // 2)
        gwb_v[j // 8, pl.ds((j % 8) * 16, 16)] = wv_v[0, j, :]
    for m in range(8):
        d0b_v[0, pl.ds(m * 16, 16)] = dsts[2 * m]
        d1b_v[0, pl.ds(m * 16, 16)] = dsts[2 * m + 1]
        tkb_v[0, pl.ds(m * 16, 16)] = wid * SPW + 8 * iota16 + m
    pltpu.sync_copy(d0b_v, d0_hbm.at[pl.ds(wid, 1)])
    pltpu.sync_copy(d1b_v, d1_hbm.at[pl.ds(wid, 1)])
    pltpu.sync_copy(tkb_v, tk_hbm.at[pl.ds(wid, 1)])
    for j in range(KTOP):
        pltpu.sync_copy(tokb_v.at[j], gtok_hbm.at[db_v.at[j]])
        pltpu.sync_copy(gwb_v.at[j], gw_hbm.at[db_v.at[j]])


@functools.lru_cache(maxsize=None)
def _dispatch_sc():
    return pl.kernel(
        _dispatch_body,
        out_type=(jax.ShapeDtypeStruct((NW, SPW), jnp.int32),
                  jax.ShapeDtypeStruct((NW, SPW), jnp.int32),
                  jax.ShapeDtypeStruct((NW, SPW), jnp.int32),
                  jax.ShapeDtypeStruct((P,), jnp.int32),
                  jax.ShapeDtypeStruct((P,), jnp.float32)),
        mesh=_sc_mesh(),
        scratch_types=[pltpu.VMEM((1, E, 16), jnp.int32),
                       pltpu.VMEM((1, 16, 16), jnp.int32),
                       pltpu.VMEM((1, 16, 16), jnp.float32),
                       pltpu.VMEM((KTOP, 128), jnp.int32),
                       pltpu.VMEM((KTOP, 128), jnp.int32),
                       pltpu.VMEM((KTOP, 128), jnp.float32),
                       pltpu.VMEM((1, SPW), jnp.int32),
                       pltpu.VMEM((1, SPW), jnp.int32),
                       pltpu.VMEM((1, SPW), jnp.int32)],
    )


def _gather_body(x_hbm, gtok_hbm, xs_hbm, idx_v, b0_v, b1_v,
                 gs0, gs1, ss0, ss1):
    wid = lax.axis_index("s") * NC + lax.axis_index("c")
    base = wid * BPW
    pltpu.sync_copy(gtok_hbm.at[pl.ds(base, BPW)], idx_v)
    for j in range(BPW // 16):
        sl = pl.ds(j * 16, 16)
        idx_v[sl] = jnp.clip(idx_v[sl], 0, T - 1)
    bufs = (b0_v, b1_v)
    gsems = (gs0, gs1)
    ssems = (ss0, ss1)

    def g_copy(k):
        b = k % 2
        return pltpu.make_async_copy(
            x_hbm.at[idx_v.at[pl.ds(k * GC, GC)]], bufs[b], gsems[b])

    def s_copy(k):
        b = k % 2
        return pltpu.make_async_copy(
            bufs[b], xs_hbm.at[pl.ds(base + k * GC, GC)], ssems[b])

    g_copy(0).start()
    for k in range(NGC):
        g_copy(k).wait()
        s_copy(k).start()
        if k + 1 < NGC:
            if k >= 1:
                s_copy(k - 1).wait()
            g_copy(k + 1).start()
    s_copy(NGC - 2).wait()
    s_copy(NGC - 1).wait()


@functools.lru_cache(maxsize=None)
def _gather():
    return pl.kernel(
        _gather_body,
        out_type=jax.ShapeDtypeStruct((P, H), jnp.float32),
        mesh=_sc_mesh(),
        scratch_types=[pltpu.VMEM((BPW,), jnp.int32),
                       pltpu.VMEM((GC, H), jnp.float32),
                       pltpu.VMEM((GC, H), jnp.float32),
                       pltpu.SemaphoreType.DMA,
                       pltpu.SemaphoreType.DMA,
                       pltpu.SemaphoreType.DMA,
                       pltpu.SemaphoreType.DMA],
    )


def _mlp_body(be_ref, nv_ref, xs_ref, wg_ref, wu_ref, wd_ref, gw_ref, y_ref):
    i = pl.program_id(0)

    @pl.when(i < nv_ref[0])
    def _():
        xb = xs_ref[...].astype(jnp.bfloat16)
        acc = jnp.zeros((BT, H), jnp.float32)
        for c in range(NFC):
            wg_c = wg_ref[0, c * FC:(c + 1) * FC, :]
            wu_c = wu_ref[0, c * FC:(c + 1) * FC, :]
            wd_c = wd_ref[0, :, c * FC:(c + 1) * FC]
            g = lax.dot_general(xb, wg_c, (((1,), (1,)), ((), ())),
                                preferred_element_type=jnp.float32)
            u = lax.dot_general(xb, wu_c, (((1,), (1,)), ((), ())),
                                preferred_element_type=jnp.float32)
            hc = (g / (1.0 + jnp.exp(-g)) * u).astype(jnp.bfloat16)
            acc = acc + lax.dot_general(hc, wd_c, (((1,), (1,)), ((), ())),
                                        preferred_element_type=jnp.float32)
        y_ref[...] = acc * gw_ref[0]


def _grouped_mlp(be, nvalid, xs, WgA, WuA, WdA, gw3):
    grid_spec = pltpu.PrefetchScalarGridSpec(
        num_scalar_prefetch=2,
        grid=(NB,),
        in_specs=[
            pl.BlockSpec((BT, H), lambda i, be, nv: (i, 0)),
            pl.BlockSpec((1, F, H), lambda i, be, nv: (be[i], 0, 0)),
            pl.BlockSpec((1, F, H), lambda i, be, nv: (be[i], 0, 0)),
            pl.BlockSpec((1, H, F), lambda i, be, nv: (be[i], 0, 0)),
            pl.BlockSpec((1, BT, 1), lambda i, be, nv: (i, 0, 0)),
        ],
        out_specs=pl.BlockSpec((BT, H), lambda i, be, nv: (i, 0)),
    )
    return pl.pallas_call(
        _mlp_body,
        grid_spec=grid_spec,
        out_shape=jax.ShapeDtypeStruct((P, H), jnp.float32),
        compiler_params=pltpu.CompilerParams(
            dimension_semantics=("arbitrary",)),
    )(be, nvalid, xs, WgA, WuA, WdA, gw3)


def _combine_body(y_hbm, ys_hbm, d0_hbm, d1_hbm, tk_hbm, out_hbm,
                  i0_v, i1_v, tk_v, a_v, b_v, c_v, o_v):
    wid = lax.axis_index("s") * NC + lax.axis_index("c")
    base = wid * RPW
    pltpu.sync_copy(d0_hbm.at[pl.ds(base, RPW)], i0_v)
    pltpu.sync_copy(d1_hbm.at[pl.ds(base, RPW)], i1_v)
    pltpu.sync_copy(tk_hbm.at[pl.ds(wid * NCC, NCC)], tk_v)
    for k in range(NCC):
        pltpu.sync_copy(y_hbm.at[i0_v.at[pl.ds(k * CC, CC)]], a_v)
        pltpu.sync_copy(y_hbm.at[i1_v.at[pl.ds(k * CC, CC)]], b_v)
        pltpu.sync_copy(ys_hbm.at[tk_v.at[k]], c_v)

        def body(r, _):
            def inner(c, _):
                sl = pl.ds(c * 16, 16)
                o_v[r, sl] = a_v[r, sl] + b_v[r, sl] + c_v[r, sl]
                return 0
            return lax.fori_loop(0, H // 16, inner, 0)
        lax.fori_loop(0, CC, body, 0)
        pltpu.sync_copy(o_v, out_hbm.at[tk_v.at[k]])


@functools.lru_cache(maxsize=None)
def _combine():
    return pl.kernel(
        _combine_body,
        out_type=jax.ShapeDtypeStruct((T, H), jnp.float32),
        mesh=_sc_mesh(),
        scratch_types=[pltpu.VMEM((RPW,), jnp.int32),
                       pltpu.VMEM((RPW,), jnp.int32),
                       pltpu.VMEM((NCC, CC), jnp.int32),
                       pltpu.VMEM((CC, H), jnp.float32),
                       pltpu.VMEM((CC, H), jnp.float32),
                       pltpu.VMEM((CC, H), jnp.float32),
                       pltpu.VMEM((CC, H), jnp.float32)],
    )


def _shared_body(x_ref, wg_ref, wu_ref, wd_ref, ys_ref):
    xb = x_ref[...].astype(jnp.bfloat16)
    acc = jnp.zeros((BTS, H), jnp.float32)
    for c in range(NFC):
        wg_c = wg_ref[c * FC:(c + 1) * FC, :]
        wu_c = wu_ref[c * FC:(c + 1) * FC, :]
        wd_c = wd_ref[:, c * FC:(c + 1) * FC]
        g = lax.dot_general(xb, wg_c, (((1,), (1,)), ((), ())),
                            preferred_element_type=jnp.float32)
        u = lax.dot_general(xb, wu_c, (((1,), (1,)), ((), ())),
                            preferred_element_type=jnp.float32)
        hc = (g / (1.0 + jnp.exp(-g)) * u).astype(jnp.bfloat16)
        acc = acc + lax.dot_general(hc, wd_c, (((1,), (1,)), ((), ())),
                                    preferred_element_type=jnp.float32)
    ys_ref[...] = acc


def _shared_mlp(x2, Wgs_b, Wus_b, Wds_b):
    return pl.pallas_call(
        _shared_body,
        grid=(T // BTS,),
        in_specs=[
            pl.BlockSpec((BTS, H), lambda i: (i, 0)),
            pl.BlockSpec((F, H), lambda i: (0, 0)),
            pl.BlockSpec((F, H), lambda i: (0, 0)),
            pl.BlockSpec((H, F), lambda i: (0, 0)),
        ],
        out_specs=pl.BlockSpec((BTS, H), lambda i: (i, 0)),
        out_shape=jax.ShapeDtypeStruct((T, H), jnp.float32),
        compiler_params=pltpu.CompilerParams(
            dimension_semantics=("arbitrary",)),
    )(x2, Wgs_b, Wus_b, Wds_b)


def kernel(x, Wr, Wg, Wu, Wd, Wgs, Wus, Wds):
    x2 = x.reshape(T, H)
    WgA = Wg.astype(jnp.bfloat16)
    WuA = Wu.astype(jnp.bfloat16)
    WdA = Wd.astype(jnp.bfloat16)
    Wgs_b = Wgs.astype(jnp.bfloat16)
    Wus_b = Wus.astype(jnp.bfloat16)
    Wds_b = Wds.astype(jnp.bfloat16)

    e2, w2, base512, o8, beP, nv = _router(x2, Wr)

    efT = e2.reshape(NW, 16, 8, KTOP).transpose(0, 2, 3, 1).reshape(NW, 16, 16)
    wfT = w2.reshape(NW, 16, 8, KTOP).transpose(0, 2, 3, 1).reshape(NW, 16, 16)
    basex2 = base512.reshape(NW, 16, E).transpose(0, 2, 1)

    d0, d1, tok, gtok, gw = _dispatch_sc()(efT, wfT, basex2)
    d0, d1 = d0.reshape(T), d1.reshape(T)

    xs = _gather()(x2, gtok)
    ys = _shared_mlp(x2, Wgs_b, Wus_b, Wds_b)
    gw3 = gw.reshape(NB, BT, 1)
    y = _grouped_mlp(beP.reshape(64), nv.reshape(1), xs, WgA, WuA, WdA, gw3)
    out = _combine()(y, ys, d0, d1, tok.reshape(T // CC, CC))
    return out.reshape(x.shape)


# speedup vs baseline: 1.3427x; 1.0053x over previous
import functools

import jax
import jax.numpy as jnp
from jax import lax
from jax.experimental import pallas as pl
from jax.experimental.pallas import tpu as pltpu
from jax.experimental.pallas import tpu_sc as plsc

T = 4096
H = 1024
F = 4096
E = 8
KTOP = 2
G = E + 1
BT = 256
P = T * KTOP + E * BT
NB = P // BT
BTS = 256
FC = 512
NFC = F // FC

NC, NS = 2, 16
NW = NC * NS
BPW = P // NW
GC = 32
NGC = BPW // GC
RPW = T // NW
CC = 16
NCC = RPW // CC

@functools.lru_cache(maxsize=None)
def _sc_mesh():
    return plsc.VectorSubcoreMesh(core_axis_name="c", subcore_axis_name="s")


CPW = (T * KTOP) // NW
SPW = T // NW


def _router_body(x_ref, wr_ref, e_ref, w_ref, base_ref, o8_ref, be_ref, nv_ref):
    logits = lax.dot_general(x_ref[...], wr_ref[...], (((1,), (1,)), ((), ())),
                             preferred_element_type=jnp.float32)
    iota = lax.broadcasted_iota(jnp.int32, logits.shape, 1)
    m0 = jnp.max(logits, axis=1, keepdims=True)
    i0 = jnp.min(jnp.where(logits == m0, iota, E), axis=1, keepdims=True)
    masked = jnp.where(iota == i0, -jnp.inf, logits)
    m1 = jnp.max(masked, axis=1, keepdims=True)
    i1 = jnp.min(jnp.where(masked == m1, iota, E), axis=1, keepdims=True)
    w0 = 1.0 / (1.0 + jnp.exp(m1 - m0))
    e_ref[...] = jnp.concatenate([i0, i1], axis=1)
    w_ref[...] = jnp.concatenate([w0, 1.0 - w0], axis=1)

    NCH = NW * 16
    ioE = lax.broadcasted_iota(jnp.int32, (T, E), 1)
    oh = ((i0 == ioE).astype(jnp.float32) + (i1 == ioE).astype(jnp.float32))
    tw = lax.broadcasted_iota(jnp.int32, (NCH, T), 0)
    tt = lax.broadcasted_iota(jnp.int32, (NCH, T), 1) // (T // NCH)
    sel = (tw == tt).astype(jnp.float32)
    ch = lax.dot_general(sel, oh, (((1,), (0,)), ((), ())),
                         preferred_element_type=jnp.float32)
    lw = lax.broadcasted_iota(jnp.int32, (NCH, NCH), 0)
    lc = lax.broadcasted_iota(jnp.int32, (NCH, NCH), 1)
    ltri = (lc < lw).astype(jnp.float32)
    pre = lax.dot_general(ltri, ch, (((1,), (0,)), ((), ())),
                          preferred_element_type=jnp.float32)
    counts = jnp.sum(ch, axis=0, keepdims=True)
    padded = jnp.ceil(counts / BT) * BT
    le = lax.broadcasted_iota(jnp.int32, (E, E), 0)
    lf = lax.broadcasted_iota(jnp.int32, (E, E), 1)
    mup = (le < lf).astype(jnp.float32)
    offsE = lax.dot_general(padded, mup, (((1,), (0,)), ((), ())),
                            preferred_element_type=jnp.float32)
    offs8 = jnp.sum(padded, axis=1, keepdims=True)
    base_ref[...] = (offsE + pre).astype(jnp.int32)
    o8_ref[...] = offs8.astype(jnp.int32)

    bstart = (lax.broadcasted_iota(jnp.int32, (1, 64), 1) * BT).astype(jnp.float32)
    ge = jnp.zeros((1, 64), jnp.float32)
    for g in range(1, E):
        ge = ge + (bstart >= offsE[0, g]).astype(jnp.float32)
    be_ref[...] = ge.astype(jnp.int32)
    nv_ref[...] = (offs8 / BT).astype(jnp.int32)


def _router(x, Wr):
    return pl.pallas_call(
        _router_body,
        out_shape=(jax.ShapeDtypeStruct((T, KTOP), jnp.int32),
                   jax.ShapeDtypeStruct((T, KTOP), jnp.float32),
                   jax.ShapeDtypeStruct((NW * 16, E), jnp.int32),
                   jax.ShapeDtypeStruct((1, 1), jnp.int32),
                   jax.ShapeDtypeStruct((1, 64), jnp.int32),
                   jax.ShapeDtypeStruct((1, 1), jnp.int32)),
    )(x, Wr)


def _dispatch_body(efT_hbm, wfT_hbm, basex_hbm,
                   d0_hbm, d1_hbm, tk_hbm, gtok_hbm, gw_hbm,
                   bx_v, ev_v, wv_v, db_v, tokb_v, gwb_v,
                   d0b_v, d1b_v, tkb_v):
    wid = lax.axis_index("s") * NC + lax.axis_index("c")
    pltpu.sync_copy(basex_hbm.at[pl.ds(wid, 1)], bx_v)
    pltpu.sync_copy(efT_hbm.at[pl.ds(wid, 1)], ev_v)
    pltpu.sync_copy(wfT_hbm.at[pl.ds(wid, 1)], wv_v)
    iota16 = lax.broadcasted_iota(jnp.int32, (16,), 0)
    one = jnp.ones((16,), jnp.int32)
    ctr = [bx_v[0, e, :] for e in range(E)]
    dsts = []
    for j in range(16):
        ev = ev_v[0, j, :]
        dst = jnp.zeros((16,), jnp.int32)
        for e in range(E):
            m = ev == e
            dst = jnp.where(m, ctr[e], dst)
            ctr[e] = ctr[e] + jnp.where(m, one, 0)
        dsts.append(dst)
        db_v[j // 8, pl.ds((j % 8) * 16, 16)] = dst
        tokb_v[j // 8, pl.ds((j % 8) * 16, 16)] = wid * SPW + 8 * iota16 + (j // 2)
        gwb_v[j // 8, pl.ds((j % 8) * 16, 16)] = wv_v[0, j, :]
    for m in range(8):
        d0b_v[0, pl.ds(m * 16, 16)] = dsts[2 * m]
        d1b_v[0, pl.ds(m * 16, 16)] = dsts[2 * m + 1]
        tkb_v[0, pl.ds(m * 16, 16)] = wid * SPW + 8 * iota16 + m
    pltpu.sync_copy(d0b_v, d0_hbm.at[pl.ds(wid, 1)])
    pltpu.sync_copy(d1b_v, d1_hbm.at[pl.ds(wid, 1)])
    pltpu.sync_copy(tkb_v, tk_hbm.at[pl.ds(wid, 1)])
    for j in range(KTOP):
        pltpu.sync_copy(tokb_v.at[j], gtok_hbm.at[db_v.at[j]])
        pltpu.sync_copy(gwb_v.at[j], gw_hbm.at[db_v.at[j]])


@functools.lru_cache(maxsize=None)
def _dispatch_sc():
    return pl.kernel(
        _dispatch_body,
        out_type=(jax.ShapeDtypeStruct((NW, SPW), jnp.int32),
                  jax.ShapeDtypeStruct((NW, SPW), jnp.int32),
                  jax.ShapeDtypeStruct((NW, SPW), jnp.int32),
                  jax.ShapeDtypeStruct((P,), jnp.int32),
                  jax.ShapeDtypeStruct((P,), jnp.float32)),
        mesh=_sc_mesh(),
        scratch_types=[pltpu.VMEM((1, E, 16), jnp.int32),
                       pltpu.VMEM((1, 16, 16), jnp.int32),
                       pltpu.VMEM((1, 16, 16), jnp.float32),
                       pltpu.VMEM((KTOP, 128), jnp.int32),
                       pltpu.VMEM((KTOP, 128), jnp.int32),
                       pltpu.VMEM((KTOP, 128), jnp.float32),
                       pltpu.VMEM((1, SPW), jnp.int32),
                       pltpu.VMEM((1, SPW), jnp.int32),
                       pltpu.VMEM((1, SPW), jnp.int32)],
    )


def _gather_body(x_hbm, gtok_hbm, xs_hbm, idx_v, b0_v, b1_v,
                 gs0, gs1, ss0, ss1):
    wid = lax.axis_index("s") * NC + lax.axis_index("c")
    base = wid * BPW
    pltpu.sync_copy(gtok_hbm.at[pl.ds(base, BPW)], idx_v)
    for j in range(BPW // 16):
        sl = pl.ds(j * 16, 16)
        idx_v[sl] = jnp.clip(idx_v[sl], 0, T - 1)
    bufs = (b0_v, b1_v)
    gsems = (gs0, gs1)
    ssems = (ss0, ss1)

    def g_copy(k):
        b = k % 2
        return pltpu.make_async_copy(
            x_hbm.at[idx_v.at[pl.ds(k * GC, GC)]], bufs[b], gsems[b])

    def s_copy(k):
        b = k % 2
        return pltpu.make_async_copy(
            bufs[b], xs_hbm.at[pl.ds(base + k * GC, GC)], ssems[b])

    g_copy(0).start()
    for k in range(NGC):
        g_copy(k).wait()
        s_copy(k).start()
        if k + 1 < NGC:
            if k >= 1:
                s_copy(k - 1).wait()
            g_copy(k + 1).start()
    s_copy(NGC - 2).wait()
    s_copy(NGC - 1).wait()


@functools.lru_cache(maxsize=None)
def _gather():
    return pl.kernel(
        _gather_body,
        out_type=jax.ShapeDtypeStruct((P, H), jnp.float32),
        mesh=_sc_mesh(),
        scratch_types=[pltpu.VMEM((BPW,), jnp.int32),
                       pltpu.VMEM((GC, H), jnp.float32),
                       pltpu.VMEM((GC, H), jnp.float32),
                       pltpu.SemaphoreType.DMA,
                       pltpu.SemaphoreType.DMA,
                       pltpu.SemaphoreType.DMA,
                       pltpu.SemaphoreType.DMA],
    )


def _mlp_body(be_ref, nv_ref, xs_ref, wg_ref, wu_ref, wd_ref, gw_ref, y_ref):
    i = pl.program_id(0)

    @pl.when(i < nv_ref[0])
    def _():
        xb = xs_ref[...].astype(jnp.bfloat16)
        acc = jnp.zeros((BT, H), jnp.float32)
        for c in range(NFC):
            wg_c = wg_ref[0, c * FC:(c + 1) * FC, :]
            wu_c = wu_ref[0, c * FC:(c + 1) * FC, :]
            wd_c = wd_ref[0, :, c * FC:(c + 1) * FC]
            g = lax.dot_general(xb, wg_c, (((1,), (1,)), ((), ())),
                                preferred_element_type=jnp.float32)
            u = lax.dot_general(xb, wu_c, (((1,), (1,)), ((), ())),
                                preferred_element_type=jnp.float32)
            hc = (g / (1.0 + jnp.exp(-g)) * u).astype(jnp.bfloat16)
            acc = acc + lax.dot_general(hc, wd_c, (((1,), (1,)), ((), ())),
                                        preferred_element_type=jnp.float32)
        y_ref[...] = acc * gw_ref[0]


def _grouped_mlp(be, nvalid, xs, WgA, WuA, WdA, gw3):
    grid_spec = pltpu.PrefetchScalarGridSpec(
        num_scalar_prefetch=2,
        grid=(NB,),
        in_specs=[
            pl.BlockSpec((BT, H), lambda i, be, nv: (i, 0)),
            pl.BlockSpec((1, F, H), lambda i, be, nv: (be[i], 0, 0)),
            pl.BlockSpec((1, F, H), lambda i, be, nv: (be[i], 0, 0)),
            pl.BlockSpec((1, H, F), lambda i, be, nv: (be[i], 0, 0)),
            pl.BlockSpec((1, BT, 1), lambda i, be, nv: (i, 0, 0)),
        ],
        out_specs=pl.BlockSpec((BT, H), lambda i, be, nv: (i, 0)),
    )
    return pl.pallas_call(
        _mlp_body,
        grid_spec=grid_spec,
        out_shape=jax.ShapeDtypeStruct((P, H), jnp.float32),
        compiler_params=pltpu.CompilerParams(
            dimension_semantics=("arbitrary",)),
    )(be, nvalid, xs, WgA, WuA, WdA, gw3)


def _combine_body(y_hbm, ys_hbm, d0_hbm, d1_hbm, tk_hbm, out_hbm,
                  i0_v, i1_v, tk_v, a_v, b_v, c_v, o_v):
    wid = lax.axis_index("s") * NC + lax.axis_index("c")
    base = wid * RPW
    pltpu.sync_copy(d0_hbm.at[pl.ds(base, RPW)], i0_v)
    pltpu.sync_copy(d1_hbm.at[pl.ds(base, RPW)], i1_v)
    pltpu.sync_copy(tk_hbm.at[pl.ds(wid * NCC, NCC)], tk_v)
    for k in range(NCC):
        pltpu.sync_copy(y_hbm.at[i0_v.at[pl.ds(k * CC, CC)]], a_v)
        pltpu.sync_copy(y_hbm.at[i1_v.at[pl.ds(k * CC, CC)]], b_v)
        pltpu.sync_copy(ys_hbm.at[tk_v.at[k]], c_v)

        def body(r, _):
            for c in range(H // 16):
                sl = pl.ds(c * 16, 16)
                o_v[r, sl] = a_v[r, sl] + b_v[r, sl] + c_v[r, sl]
            return 0
        lax.fori_loop(0, CC, body, 0)
        pltpu.sync_copy(o_v, out_hbm.at[tk_v.at[k]])


@functools.lru_cache(maxsize=None)
def _combine():
    return pl.kernel(
        _combine_body,
        out_type=jax.ShapeDtypeStruct((T, H), jnp.float32),
        mesh=_sc_mesh(),
        scratch_types=[pltpu.VMEM((RPW,), jnp.int32),
                       pltpu.VMEM((RPW,), jnp.int32),
                       pltpu.VMEM((NCC, CC), jnp.int32),
                       pltpu.VMEM((CC, H), jnp.float32),
                       pltpu.VMEM((CC, H), jnp.float32),
                       pltpu.VMEM((CC, H), jnp.float32),
                       pltpu.VMEM((CC, H), jnp.float32)],
    )


def _shared_body(x_ref, wg_ref, wu_ref, wd_ref, ys_ref):
    xb = x_ref[...].astype(jnp.bfloat16)
    acc = jnp.zeros((BTS, H), jnp.float32)
    for c in range(NFC):
        wg_c = wg_ref[c * FC:(c + 1) * FC, :]
        wu_c = wu_ref[c * FC:(c + 1) * FC, :]
        wd_c = wd_ref[:, c * FC:(c + 1) * FC]
        g = lax.dot_general(xb, wg_c, (((1,), (1,)), ((), ())),
                            preferred_element_type=jnp.float32)
        u = lax.dot_general(xb, wu_c, (((1,), (1,)), ((), ())),
                            preferred_element_type=jnp.float32)
        hc = (g / (1.0 + jnp.exp(-g)) * u).astype(jnp.bfloat16)
        acc = acc + lax.dot_general(hc, wd_c, (((1,), (1,)), ((), ())),
                                    preferred_element_type=jnp.float32)
    ys_ref[...] = acc


def _shared_mlp(x2, Wgs_b, Wus_b, Wds_b):
    return pl.pallas_call(
        _shared_body,
        grid=(T // BTS,),
        in_specs=[
            pl.BlockSpec((BTS, H), lambda i: (i, 0)),
            pl.BlockSpec((F, H), lambda i: (0, 0)),
            pl.BlockSpec((F, H), lambda i: (0, 0)),
            pl.BlockSpec((H, F), lambda i: (0, 0)),
        ],
        out_specs=pl.BlockSpec((BTS, H), lambda i: (i, 0)),
        out_shape=jax.ShapeDtypeStruct((T, H), jnp.float32),
        compiler_params=pltpu.CompilerParams(
            dimension_semantics=("arbitrary",)),
    )(x2, Wgs_b, Wus_b, Wds_b)


def kernel(x, Wr, Wg, Wu, Wd, Wgs, Wus, Wds):
    x2 = x.reshape(T, H)
    WgA = Wg.astype(jnp.bfloat16)
    WuA = Wu.astype(jnp.bfloat16)
    WdA = Wd.astype(jnp.bfloat16)
    Wgs_b = Wgs.astype(jnp.bfloat16)
    Wus_b = Wus.astype(jnp.bfloat16)
    Wds_b = Wds.astype(jnp.bfloat16)

    e2, w2, base512, o8, beP, nv = _router(x2, Wr)

    efT = e2.reshape(NW, 16, 8, KTOP).transpose(0, 2, 3, 1).reshape(NW, 16, 16)
    wfT = w2.reshape(NW, 16, 8, KTOP).transpose(0, 2, 3, 1).reshape(NW, 16, 16)
    basex2 = base512.reshape(NW, 16, E).transpose(0, 2, 1)

    d0, d1, tok, gtok, gw = _dispatch_sc()(efT, wfT, basex2)
    d0, d1 = d0.reshape(T), d1.reshape(T)

    xs = _gather()(x2, gtok)
    ys = _shared_mlp(x2, Wgs_b, Wus_b, Wds_b)
    gw3 = gw.reshape(NB, BT, 1)
    y = _grouped_mlp(beP.reshape(64), nv.reshape(1), xs, WgA, WuA, WdA, gw3)
    out = _combine()(y, ys, d0, d1, tok.reshape(T // CC, CC))
    return out.reshape(x.shape)
